# baseline (speedup 1.0000x reference)
import jax
import jax.numpy as jnp
from jax.experimental import pallas as pl
from jax.experimental.pallas import tpu as pltpu

_N = 4096
_BLK = 512
_NB = _N // _BLK
_Q = 4
_QROWS = _N // _Q
_QBLK = _QROWS // _NB


def _bf(x):
    return x.astype(jnp.bfloat16)


def _dot(a, b):
    return jax.lax.dot_general(
        a, b, (((1,), (0,)), ((), ())), preferred_element_type=jnp.float32)


def _prelu(x, a):
    return jnp.where(x >= 0, x, a * x)


def _fused_gcn_kernel(
    adj0_ref, adj1_ref, adj2_ref, adj3_ref, h2_ref,
    w0d_ref, b0_ref, a0_ref,
    w1d_ref, b1_ref, a1_ref,
    wd0_ref, bd0_ref, ad0_ref,
    wd1_ref, bd1_ref, ad1_ref,
    en0_ref, sh0_ref, es1_ref, de0_ref, de1_ref,
    adj_vmem, fts_ref,
):
    s = pl.program_id(0)
    i = pl.program_id(1)

    @pl.when((s == 0) & (i == 0))
    def _():
        fts_ref[...] = _bf(_dot(h2_ref[...], w0d_ref[...]))

    @pl.when((s == 1) & (i == 0))
    def _():
        x = jnp.concatenate([_bf(en0_ref[...]), _bf(sh0_ref[...])], axis=1)
        fts_ref[:, :128] = _bf(_dot(x, w1d_ref[...]))

    @pl.when((s == 2) & (i == 0))
    def _():
        fts_ref[:, :128] = _bf(_dot(_bf(es1_ref[...]), wd0_ref[...]))

    @pl.when((s == 3) & (i == 0))
    def _():
        fts_ref[:, :128] = _bf(_dot(_bf(de0_ref[...]), wd1_ref[...]))

    @pl.when(s == 0)
    def _():
        a0 = a0_ref[0, 0]
        b0 = b0_ref[...]
        for q, aq_ref in enumerate((adj0_ref, adj1_ref, adj2_ref, adj3_ref)):
            rows_q = pl.ds(q * _QROWS + i * _QBLK, _QBLK)
            blk = _bf(aq_ref[0])
            adj_vmem[rows_q, :] = blk
            z = _dot(blk, fts_ref[...])
            en0_ref[rows_q, :] = _prelu(z[:, :128] + b0, a0)
            sh0_ref[rows_q, :] = _prelu(z[:, 128:] + b0, a0)

    rows = pl.ds(i * _BLK, _BLK)

    @pl.when(s == 1)
    def _():
        z = _dot(adj_vmem[rows, :], fts_ref[:, :128])
        es1_ref[rows, :] = _prelu(z + b1_ref[...], a1_ref[0, 0])

    @pl.when(s == 2)
    def _():
        z = _dot(adj_vmem[rows, :], fts_ref[:, :128])
        de0_ref[rows, :] = _prelu(z + bd0_ref[...], ad0_ref[0, 0])

    @pl.when(s == 3)
    def _():
        z = _dot(adj_vmem[rows, :], fts_ref[:, :128])
        de1_ref[...] = _prelu(z + bd1_ref[...], ad1_ref[0, 0])


def kernel(h, shuf_h, mps,
           enc0_W, enc0_b, enc0_a,
           enc1_W, enc1_b, enc1_a,
           dec0_W, dec0_b, dec0_a,
           dec1_W, dec1_b, dec1_a):
    f32 = jnp.float32
    bf16 = jnp.bfloat16
    adj4 = mps[0].reshape(_Q, _QROWS, _N)

    h2 = jnp.concatenate([h, shuf_h], axis=1).astype(bf16)
    w0d = jnp.zeros((256, 256), f32).at[:128, :128].set(enc0_W) \
                                    .at[128:, 128:].set(enc0_W).astype(bf16)
    w1d = jnp.zeros((256, 128), f32).at[:128, :64].set(enc1_W) \
                                    .at[128:, 64:].set(enc1_W).astype(bf16)
    b1c = jnp.concatenate([enc1_b, enc1_b]).reshape(1, -1)
    wd0 = jnp.zeros((128, 128), f32).at[:64, :].set(dec0_W).astype(bf16)
    wd1 = dec1_W.astype(bf16)

    out_shapes = [
        jax.ShapeDtypeStruct((_N, 128), f32),
        jax.ShapeDtypeStruct((_N, 128), f32),
        jax.ShapeDtypeStruct((_N, 128), f32),
        jax.ShapeDtypeStruct((_N, 128), f32),
        jax.ShapeDtypeStruct((_N, 128), f32),
    ]

    inputs = (
        adj4, adj4, adj4, adj4, h2,
        w0d, enc0_b.reshape(1, -1), enc0_a.reshape(1, 1),
        w1d, b1c, enc1_a.reshape(1, 1),
        wd0, dec0_b.reshape(1, -1), dec0_a.reshape(1, 1),
        wd1, dec1_b.reshape(1, -1), dec1_a.reshape(1, 1),
    )

    def _full(shape):
        nd = len(shape)
        return pl.BlockSpec(shape, lambda s, i, _nd=nd: (0,) * _nd)

    def _adj_spec(q):
        return pl.BlockSpec(
            (1, _QBLK, _N),
            lambda s, i, _q=q: (_q, jnp.where(s == 0, i, _NB - 1), 0))

    in_specs = [_adj_spec(q) for q in range(_Q)] + \
               [_full(x.shape) for x in inputs[_Q:]]
    de1_spec = pl.BlockSpec(
        (_BLK, 128), lambda s, i: (jnp.where(s == 3, i, 0), 0))
    out_specs = [_full(sh.shape) for sh in out_shapes[:-1]] + [de1_spec]

    en0, sh0, es1, de0, de1 = pl.pallas_call(
        _fused_gcn_kernel,
        grid=(4, _NB),
        out_shape=out_shapes,
        in_specs=in_specs,
        out_specs=out_specs,
        scratch_shapes=[
            pltpu.VMEM((_N, _N), jnp.bfloat16),
            pltpu.VMEM((_N, 256), jnp.bfloat16),
        ],
        compiler_params=pltpu.CompilerParams(
            dimension_semantics=("arbitrary", "arbitrary"),
            vmem_limit_bytes=63 * 1024 * 1024,
        ),
    )(*inputs)

    en1 = es1[:, :64]
    sh1 = es1[:, 64:]
    return (en1, (en0, en1), sh1, (sh0, sh1), de1, (de0, de1))


# speedup vs baseline: 1.0373x; 1.0373x over previous
import jax
import jax.numpy as jnp
from jax.experimental import pallas as pl
from jax.experimental.pallas import tpu as pltpu

_N = 4096
_BLK = 512
_NB = _N // _BLK
_CH = 128
_NCH = _N // _CH
_CPB = _BLK // _CH
_DEPTH = 6


def _bf(x):
    return x.astype(jnp.bfloat16)


def _dot(a, b):
    return jax.lax.dot_general(
        a, b, (((1,), (0,)), ((), ())), preferred_element_type=jnp.float32)


def _prelu(x, a):
    return jnp.where(x >= 0, x, a * x)


def _fused_gcn_kernel(
    adj_hbm, h2_ref,
    w0d_ref, b0_ref, a0_ref,
    w1d_ref, b1_ref, a1_ref,
    wd0_ref, bd0_ref, ad0_ref,
    wd1_ref, bd1_ref, ad1_ref,
    en0_ref, sh0_ref, es1_ref, de0_ref, de1_ref,
    adj_vmem, fts_ref, stage_buf, sems,
):
    s = pl.program_id(0)
    i = pl.program_id(1)

    def _chunk_copy(c, slot, priority):
        return pltpu.make_async_copy(
            adj_hbm.at[pl.ds(c * _CH, _CH), :], stage_buf.at[slot],
            sems.at[slot]), priority

    @pl.when((s == 0) & (i == 0))
    def _():
        for c in range(_DEPTH):
            desc, prio = _chunk_copy(c, c, c % 2)
            desc.start(priority=prio)
        fts_ref[...] = _bf(_dot(h2_ref[...], w0d_ref[...]))

    @pl.when((s == 1) & (i == 0))
    def _():
        x = jnp.concatenate([_bf(en0_ref[...]), _bf(sh0_ref[...])], axis=1)
        fts_ref[:, :128] = _bf(_dot(x, w1d_ref[...]))

    @pl.when((s == 2) & (i == 0))
    def _():
        fts_ref[:, :128] = _bf(_dot(_bf(es1_ref[...]), wd0_ref[...]))

    @pl.when((s == 3) & (i == 0))
    def _():
        fts_ref[:, :128] = _bf(_dot(_bf(de0_ref[...]), wd1_ref[...]))

    rows = pl.ds(i * _BLK, _BLK)

    @pl.when(s == 0)
    def _():
        for k in range(_CPB):
            c = i * _CPB + k
            slot = jax.lax.rem(c, _DEPTH)
            pltpu.make_async_copy(
                adj_hbm.at[pl.ds(c * _CH, _CH), :], stage_buf.at[slot],
                sems.at[slot]).wait()
            adj_vmem[pl.ds(c * _CH, _CH), :] = _bf(stage_buf[slot])
            nc = c + _DEPTH

            @pl.when(nc < _NCH)
            def _():
                nslot = jax.lax.rem(nc, _DEPTH)
                desc = pltpu.make_async_copy(
                    adj_hbm.at[pl.ds(nc * _CH, _CH), :], stage_buf.at[nslot],
                    sems.at[nslot])
                desc.start(priority=k % 2)

        z = _dot(adj_vmem[rows, :], fts_ref[...])
        a0 = a0_ref[0, 0]
        b0 = b0_ref[...]
        en0_ref[rows, :] = _prelu(z[:, :128] + b0, a0)
        sh0_ref[rows, :] = _prelu(z[:, 128:] + b0, a0)

    @pl.when(s == 1)
    def _():
        z = _dot(adj_vmem[rows, :], fts_ref[:, :128])
        es1_ref[rows, :] = _prelu(z + b1_ref[...], a1_ref[0, 0])

    @pl.when(s == 2)
    def _():
        z = _dot(adj_vmem[rows, :], fts_ref[:, :128])
        de0_ref[rows, :] = _prelu(z + bd0_ref[...], ad0_ref[0, 0])

    @pl.when(s == 3)
    def _():
        z = _dot(adj_vmem[rows, :], fts_ref[:, :128])
        de1_ref[...] = _prelu(z + bd1_ref[...], ad1_ref[0, 0])


def kernel(h, shuf_h, mps,
           enc0_W, enc0_b, enc0_a,
           enc1_W, enc1_b, enc1_a,
           dec0_W, dec0_b, dec0_a,
           dec1_W, dec1_b, dec1_a):
    f32 = jnp.float32
    bf16 = jnp.bfloat16
    adj = mps[0]

    h2 = jnp.concatenate([h, shuf_h], axis=1).astype(bf16)
    w0d = jnp.zeros((256, 256), f32).at[:128, :128].set(enc0_W) \
                                    .at[128:, 128:].set(enc0_W).astype(bf16)
    w1d = jnp.zeros((256, 128), f32).at[:128, :64].set(enc1_W) \
                                    .at[128:, 64:].set(enc1_W).astype(bf16)
    b1c = jnp.concatenate([enc1_b, enc1_b]).reshape(1, -1)
    wd0 = jnp.zeros((128, 128), f32).at[:64, :].set(dec0_W).astype(bf16)
    wd1 = dec1_W.astype(bf16)

    out_shapes = [
        jax.ShapeDtypeStruct((_N, 128), f32),
        jax.ShapeDtypeStruct((_N, 128), f32),
        jax.ShapeDtypeStruct((_N, 128), f32),
        jax.ShapeDtypeStruct((_N, 128), f32),
        jax.ShapeDtypeStruct((_N, 128), f32),
    ]

    inputs = (
        adj, h2,
        w0d, enc0_b.reshape(1, -1), enc0_a.reshape(1, 1),
        w1d, b1c, enc1_a.reshape(1, 1),
        wd0, dec0_b.reshape(1, -1), dec0_a.reshape(1, 1),
        wd1, dec1_b.reshape(1, -1), dec1_a.reshape(1, 1),
    )

    def _full(shape):
        nd = len(shape)
        return pl.BlockSpec(shape, lambda s, i, _nd=nd: (0,) * _nd)

    in_specs = [pl.BlockSpec(memory_space=pltpu.MemorySpace.HBM)] + \
               [_full(x.shape) for x in inputs[1:]]
    de1_spec = pl.BlockSpec(
        (_BLK, 128), lambda s, i: (jnp.where(s == 3, i, 0), 0))
    out_specs = [_full(sh.shape) for sh in out_shapes[:-1]] + [de1_spec]

    en0, sh0, es1, de0, de1 = pl.pallas_call(
        _fused_gcn_kernel,
        grid=(4, _NB),
        out_shape=out_shapes,
        in_specs=in_specs,
        out_specs=out_specs,
        scratch_shapes=[
            pltpu.VMEM((_N, _N), jnp.bfloat16),
            pltpu.VMEM((_N, 256), jnp.bfloat16),
            pltpu.VMEM((_DEPTH, _CH, _N), jnp.float32),
            pltpu.SemaphoreType.DMA((_DEPTH,)),
        ],
        compiler_params=pltpu.CompilerParams(
            dimension_semantics=("arbitrary", "arbitrary"),
            vmem_limit_bytes=63 * 1024 * 1024,
        ),
    )(*inputs)

    en1 = es1[:, :64]
    sh1 = es1[:, 64:]
    return (en1, (en0, en1), sh1, (sh0, sh1), de1, (de0, de1))
